# baseline (speedup 1.0000x reference)
import functools

import jax
import jax.numpy as jnp
from jax import lax
from jax.experimental import pallas as pl
from jax.experimental.pallas import tpu as pltpu
from jax.experimental.pallas import tpu_sc as plsc

N = 100000
D = 128
S = 64

NC = 2
NS = 16
NW = NC * NS

B = 160
NB = N // B
CH = 80
NCH = B // CH
G = B // 16
RD = 4
NBJ = 5

_MESH = plsc.VectorSubcoreMesh(
    core_axis_name="c", subcore_axis_name="s", num_cores=NC, num_subcores=NS
)


@functools.partial(
    pl.kernel,
    out_type=jax.ShapeDtypeStruct((NW, S, D), jnp.float32),
    mesh=_MESH,
    compiler_params=pltpu.CompilerParams(needs_layout_passes=False),
    scratch_types=[
        [pltpu.VMEM((B, D), jnp.float32) for _ in range(RD)],
        [pltpu.VMEM((NCH, CH), jnp.int32) for _ in range(RD)],
        pltpu.VMEM((D,), jnp.float32),
        pltpu.VMEM((D,), jnp.float32),
        pltpu.VMEM((16,), jnp.float32),
        pltpu.VMEM((B, 16), jnp.float32),
        pltpu.VMEM((S, D), jnp.float32),
        [pltpu.SemaphoreType.DMA for _ in range(RD)],
    ],
)
def _sc_pool(x_hbm, idx_hbm, wa_hbm, wm_hbm, b_hbm, out_hbm,
             xb, ib, wav, wmv, bv, sbuf, accl, semx):
    cid = lax.axis_index("c")
    sid = lax.axis_index("s")
    wid = sid * NC + cid

    pltpu.sync_copy(wa_hbm, wav)
    pltpu.sync_copy(wm_hbm, wmv)
    pltpu.sync_copy(b_hbm, bv)

    zeros16 = jnp.zeros((16,), jnp.float32)

    def zbody(i, carry):
        accl[i // (D // 16), pl.ds((i % (D // 16)) * 16, 16)] = zeros16
        return carry

    lax.fori_loop(0, S * (D // 16), zbody, 0)

    wa_k = [wav[pl.ds(k * 16, 16)] for k in range(D // 16)]
    wm_k = [wmv[pl.ds(k * 16, 16)] for k in range(D // 16)]
    bvec = bv[...]
    ba = bvec[0]
    bm = bvec[1]

    nb_w = (NB - wid + NW - 1) // NW

    def _compute(xsrc, isrc):
        @plsc.parallel_loop(0, B, unroll=8)
        def _score(r):
            x0 = xsrc[r, pl.ds(0, 16)]
            pa = x0 * wa_k[0]
            pm = x0 * wm_k[0]
            for k in range(1, D // 16):
                xk = xsrc[r, pl.ds(k * 16, 16)]
                pa = pa + xk * wa_k[k]
                pm = pm + xk * wm_k[k]
            pa_s = jnp.sum(pa) + ba
            pm_s = jnp.sum(pm) + bm
            sv = jnp.full((16,), pa_s, jnp.float32)
            mv = jnp.full((16,), pm_s, jnp.float32)
            sbuf[r, :] = sv / (1.0 + jnp.exp(-mv))

        def gbody(g, carry):
            iv = isrc[g // (CH // 16), pl.ds((g % (CH // 16)) * 16, 16)]
            for i in range(16):
                r = g * 16 + i
                seg = iv[i]
                w = sbuf[r, :]
                for k in range(D // 16):
                    plsc.addupdate(
                        accl.at[seg, pl.ds(k * 16, 16)],
                        xsrc[r, pl.ds(k * 16, 16)] * w,
                    )
            return carry

        lax.fori_loop(0, G, gbody, 0)

    def _start_in(blk, t):
        pltpu.async_copy(x_hbm.at[pl.ds(blk * B, B)], xb[t], semx[t])
        pltpu.async_copy(idx_hbm.at[pl.ds(blk * NCH, NCH)], ib[t], semx[t])

    def _wait_in(blk, t):
        pltpu.make_async_copy(
            x_hbm.at[pl.ds(blk * B, B)], xb[t], semx[t]).wait()
        pltpu.make_async_copy(
            idx_hbm.at[pl.ds(blk * NCH, NCH)], ib[t], semx[t]).wait()

    for p in range(3):
        _start_in(wid + p * NW, p)

    def outer(jj, carry):
        for t in range(RD):
            j = jj * RD + t
            b = wid + j * NW
            pf = j + 3
            bp = wid + pf * NW
            n = (t + 3) % RD

            @pl.when(pf < nb_w)
            def _prefetch():
                _start_in(bp, n)

            @pl.when(j < nb_w)
            def _work():
                _wait_in(b, t)
                _compute(xb[t], ib[t])
        return carry

    lax.fori_loop(0, NBJ, outer, 0)

    pltpu.sync_copy(accl, out_hbm.at[wid])


def _combine_body(p_ref, o_ref):
    o_ref[...] = jnp.sum(p_ref[...], axis=0)


_combine = pl.pallas_call(
    _combine_body,
    out_shape=jax.ShapeDtypeStruct((S, D), jnp.float32),
)


@jax.jit
def kernel(node_feats, batch_idx, W_attn, b_attn, W_mask, b_mask):
    idx = batch_idx.astype(jnp.int32).reshape(N // CH, CH)
    wa = W_attn.reshape(D)
    wm = W_mask.reshape(D)
    bias = jnp.concatenate(
        [b_attn.astype(jnp.float32), b_mask.astype(jnp.float32),
         jnp.zeros((14,), jnp.float32)]
    )
    partials = _sc_pool(node_feats, idx, wa, wm, bias)
    return _combine(partials)


# speedup vs baseline: 2.2057x; 2.2057x over previous
import functools

import jax
import jax.numpy as jnp
from jax import lax
from jax.experimental import pallas as pl
from jax.experimental.pallas import tpu as pltpu
from jax.experimental.pallas import tpu_sc as plsc

N = 100000
D = 128
S = 64

NC = 2
NS = 16
NW = NC * NS

B = 160
NB = N // B
CH = 80
NCH = B // CH
NBJ = 5

_MESH = plsc.VectorSubcoreMesh(
    core_axis_name="c", subcore_axis_name="s", num_cores=NC, num_subcores=NS
)


@functools.partial(
    pl.kernel,
    out_type=jax.ShapeDtypeStruct((NC, S, D), jnp.float32),
    mesh=_MESH,
    compiler_params=pltpu.CompilerParams(needs_layout_passes=False),
    scratch_types=[
        [pltpu.VMEM((B, D), jnp.float32) for _ in range(4)],
        [pltpu.VMEM((NCH, CH), jnp.int32) for _ in range(4)],
        pltpu.VMEM((D,), jnp.float32),
        pltpu.VMEM((D,), jnp.float32),
        pltpu.VMEM((16,), jnp.float32),
        pltpu.VMEM((B, 16), jnp.float32),
        [pltpu.VMEM((16, D), jnp.float32) for _ in range(4)],
        pltpu.VMEM((S, D), jnp.float32),
        [pltpu.SemaphoreType.DMA for _ in range(4)],
        [pltpu.SemaphoreType.DMA for _ in range(4)],
        pltpu.VMEM_SHARED((S, D), jnp.float32),
    ],
)
def _sc_pool(x_hbm, idx_hbm, wa_hbm, wm_hbm, b_hbm, out_hbm,
             xb, ib, wav, wmv, bv, sbuf, pb, zbuf, semx, sems, acc):
    cid = lax.axis_index("c")
    sid = lax.axis_index("s")
    wid = sid * NC + cid

    pltpu.sync_copy(wa_hbm, wav)
    pltpu.sync_copy(wm_hbm, wmv)
    pltpu.sync_copy(b_hbm, bv)

    @pl.when(sid == 0)
    def _init():
        zeros16 = jnp.zeros((16,), jnp.float32)

        def zbody(i, carry):
            zbuf[i // (D // 16), pl.ds((i % (D // 16)) * 16, 16)] = zeros16
            return carry

        lax.fori_loop(0, S * (D // 16), zbody, 0)
        pltpu.sync_copy(zbuf, acc)

    plsc.subcore_barrier()

    zz = jnp.zeros((16,), jnp.float32)

    def pzbody(i, carry):
        q = i // (15 * (D // 16))
        rr = i % (15 * (D // 16))
        pb[0][1 + rr // (D // 16), pl.ds((rr % (D // 16)) * 16, 16)] = zz
        return carry

    for q in range(4):
        def pzb(i, carry, _q=q):
            pb[_q][1 + i // (D // 16), pl.ds((i % (D // 16)) * 16, 16)] = zz
            return carry
        lax.fori_loop(0, 15 * (D // 16), pzb, 0)

    wa_k = [wav[pl.ds(k * 16, 16)] for k in range(D // 16)]
    wm_k = [wmv[pl.ds(k * 16, 16)] for k in range(D // 16)]
    bvec = bv[...]
    ba = bvec[0]
    bm = bvec[1]

    nb_w = (NB - wid + NW - 1) // NW

    def _score_pass(xsrc):
        @plsc.parallel_loop(0, B, unroll=8)
        def _score(r):
            x0 = xsrc[r, pl.ds(0, 16)]
            pa = x0 * wa_k[0]
            pm = x0 * wm_k[0]
            for k in range(1, D // 16):
                xk = xsrc[r, pl.ds(k * 16, 16)]
                pa = pa + xk * wa_k[k]
                pm = pm + xk * wm_k[k]
            pa_s = jnp.sum(pa) + ba
            pm_s = jnp.sum(pm) + bm
            sv = jnp.full((16,), pa_s, jnp.float32)
            mv = jnp.full((16,), pm_s, jnp.float32)
            sbuf[r, :] = sv / (1.0 + jnp.exp(-mv))

    def _scale_pass(xsrc):
        @plsc.parallel_loop(0, B, unroll=8)
        def _scale(r):
            w = sbuf[r, :]
            for k in range(D // 16):
                xsrc[r, pl.ds(k * 16, 16)] = xsrc[r, pl.ds(k * 16, 16)] * w

    def _partial_pass(xsrc, pdst):
        zcar = tuple(jnp.zeros((16,), jnp.float32) for _ in range(D // 16))

        @plsc.parallel_loop(0, B, unroll=4, carry=zcar)
        def _acc(r, c):
            w = sbuf[r, :]
            return tuple(
                c[k] + xsrc[r, pl.ds(k * 16, 16)] * w
                for k in range(D // 16)
            )

        for k in range(D // 16):
            pdst[0, pl.ds(k * 16, 16)] = _acc[k]

    def _uniform(t4):
        iv0 = ib[t4][0, pl.ds(0, 16)]
        ivL = ib[t4][NCH - 1, pl.ds(CH - 16, 16)]
        return iv0, iv0[0] == ivL[15]

    def _start_in(blk, t4):
        pltpu.async_copy(x_hbm.at[pl.ds(blk * B, B)], xb[t4], semx[t4])
        pltpu.async_copy(idx_hbm.at[pl.ds(blk * NCH, NCH)], ib[t4], semx[t4])

    def _wait_in(blk, t4):
        pltpu.make_async_copy(
            x_hbm.at[pl.ds(blk * B, B)], xb[t4], semx[t4]).wait()
        pltpu.make_async_copy(
            idx_hbm.at[pl.ds(blk * NCH, NCH)], ib[t4], semx[t4]).wait()

    def _wait_scatter(t4):
        iv0, uni = _uniform(t4)

        @pl.when(uni)
        def _wu():
            pltpu.make_async_copy(pb[t4], acc.at[iv0], sems[t4]).wait()

        @pl.when(jnp.logical_not(uni))
        def _wm():
            for c in range(NCH):
                pltpu.make_async_copy(
                    xb[t4].at[pl.ds(c * CH, CH)], acc.at[ib[t4].at[c]],
                    sems[t4]).wait()

    _start_in(wid, 0)

    def outer(jj, carry):
        for t in range(4):
            j = jj * 4 + t
            b = wid + j * NW
            nxt = j + 1
            bn = wid + nxt * NW
            n4 = (t + 1) % 4

            @pl.when((j >= 3) & (nxt < nb_w))
            def _free_next():
                _wait_scatter(n4)

            @pl.when(nxt < nb_w)
            def _prefetch():
                _start_in(bn, n4)

            @pl.when(j < nb_w)
            def _work():
                _wait_in(b, t)
                _score_pass(xb[t])
                iv0, uni = _uniform(t)

                @pl.when(uni)
                def _wu():
                    _partial_pass(xb[t], pb[t])
                    pltpu.async_copy(pb[t], acc.at[iv0], sems[t], add=True)

                @pl.when(jnp.logical_not(uni))
                def _wm():
                    _scale_pass(xb[t])
                    for c in range(NCH):
                        pltpu.async_copy(
                            xb[t].at[pl.ds(c * CH, CH)], acc.at[ib[t].at[c]],
                            sems[t], add=True)
        return carry

    lax.fori_loop(0, NBJ, outer, 0)

    for p in range(4):
        _wait_scatter(p)

    plsc.subcore_barrier()

    @pl.when(sid == 0)
    def _writeout():
        pltpu.sync_copy(acc, out_hbm.at[cid])


def _combine_body(p_ref, o_ref):
    o_ref[...] = p_ref[0] + p_ref[1]


_combine = pl.pallas_call(
    _combine_body,
    out_shape=jax.ShapeDtypeStruct((S, D), jnp.float32),
)


@jax.jit
def kernel(node_feats, batch_idx, W_attn, b_attn, W_mask, b_mask):
    idx = batch_idx.astype(jnp.int32).reshape(N // CH, CH)
    wa = W_attn.reshape(D)
    wm = W_mask.reshape(D)
    bias = jnp.concatenate(
        [b_attn.astype(jnp.float32), b_mask.astype(jnp.float32),
         jnp.zeros((14,), jnp.float32)]
    )
    partials = _sc_pool(node_feats, idx, wa, wm, bias)
    return _combine(partials)


# speedup vs baseline: 2.3063x; 1.0456x over previous
import functools

import jax
import jax.numpy as jnp
from jax import lax
from jax.experimental import pallas as pl
from jax.experimental.pallas import tpu as pltpu
from jax.experimental.pallas import tpu_sc as plsc

N = 100000
D = 128
S = 64

NC = 2
NS = 16
NW = NC * NS

B = 160
NB = N // B
CH = 80
NCH = B // CH
NBJ = 5

_MESH = plsc.VectorSubcoreMesh(
    core_axis_name="c", subcore_axis_name="s", num_cores=NC, num_subcores=NS
)


@functools.partial(
    pl.kernel,
    out_type=jax.ShapeDtypeStruct((NC, S, D), jnp.float32),
    mesh=_MESH,
    compiler_params=pltpu.CompilerParams(needs_layout_passes=False),
    scratch_types=[
        [pltpu.VMEM((B, D), jnp.float32) for _ in range(4)],
        [pltpu.VMEM((NCH, CH), jnp.int32) for _ in range(4)],
        pltpu.VMEM((D,), jnp.float32),
        pltpu.VMEM((D,), jnp.float32),
        pltpu.VMEM((16,), jnp.float32),
        pltpu.VMEM((B, 16), jnp.float32),
        pltpu.VMEM((S, D), jnp.float32),
        [pltpu.SemaphoreType.DMA for _ in range(4)],
        [pltpu.SemaphoreType.DMA for _ in range(4)],
        pltpu.VMEM_SHARED((S, D), jnp.float32),
    ],
)
def _sc_pool(x_hbm, idx_hbm, wa_hbm, wm_hbm, b_hbm, out_hbm,
             xb, ib, wav, wmv, bv, sbuf, zbuf, semx, sems, acc):
    cid = lax.axis_index("c")
    sid = lax.axis_index("s")
    wid = sid * NC + cid

    pltpu.sync_copy(wa_hbm, wav)
    pltpu.sync_copy(wm_hbm, wmv)
    pltpu.sync_copy(b_hbm, bv)

    @pl.when(sid == 0)
    def _init():
        zeros16 = jnp.zeros((16,), jnp.float32)

        def zbody(i, carry):
            zbuf[i // (D // 16), pl.ds((i % (D // 16)) * 16, 16)] = zeros16
            return carry

        lax.fori_loop(0, S * (D // 16), zbody, 0)
        pltpu.sync_copy(zbuf, acc)

    plsc.subcore_barrier()

    wa_k = [wav[pl.ds(k * 16, 16)] for k in range(D // 16)]
    wm_k = [wmv[pl.ds(k * 16, 16)] for k in range(D // 16)]
    bvec = bv[...]
    ba = bvec[0]
    bm = bvec[1]

    nb_w = (NB - wid + NW - 1) // NW

    def _compute(xsrc):
        @plsc.parallel_loop(0, B, unroll=8)
        def _score(r):
            x0 = xsrc[r, pl.ds(0, 16)]
            pa = x0 * wa_k[0]
            pm = x0 * wm_k[0]
            for k in range(1, D // 16):
                xk = xsrc[r, pl.ds(k * 16, 16)]
                pa = pa + xk * wa_k[k]
                pm = pm + xk * wm_k[k]
            pa_s = jnp.sum(pa) + ba
            pm_s = jnp.sum(pm) + bm
            sv = jnp.full((16,), pa_s, jnp.float32)
            mv = jnp.full((16,), pm_s, jnp.float32)
            sbuf[r, :] = sv / (1.0 + jnp.exp(-mv))

        @plsc.parallel_loop(0, B, unroll=8)
        def _scale(r):
            w = sbuf[r, :]
            for k in range(D // 16):
                xsrc[r, pl.ds(k * 16, 16)] = xsrc[r, pl.ds(k * 16, 16)] * w

    def _start_in(blk, t4):
        pltpu.async_copy(x_hbm.at[pl.ds(blk * B, B)], xb[t4], semx[t4])
        pltpu.async_copy(idx_hbm.at[pl.ds(blk * NCH, NCH)], ib[t4], semx[t4])

    def _wait_in(blk, t4):
        pltpu.make_async_copy(
            x_hbm.at[pl.ds(blk * B, B)], xb[t4], semx[t4]).wait()
        pltpu.make_async_copy(
            idx_hbm.at[pl.ds(blk * NCH, NCH)], ib[t4], semx[t4]).wait()

    def _wait_scatter(t4):
        for c in range(NCH):
            pltpu.make_async_copy(
                xb[t4].at[pl.ds(c * CH, CH)], acc.at[ib[t4].at[c]],
                sems[t4]).wait()

    _start_in(wid, 0)

    def outer(jj, carry):
        for t in range(4):
            j = jj * 4 + t
            b = wid + j * NW
            nxt = j + 1
            bn = wid + nxt * NW
            n4 = (t + 1) % 4

            @pl.when((j >= 3) & (nxt < nb_w))
            def _free_next():
                _wait_scatter(n4)

            @pl.when(nxt < nb_w)
            def _prefetch():
                _start_in(bn, n4)

            @pl.when(j < nb_w)
            def _work():
                _wait_in(b, t)
                _compute(xb[t])
                for c in range(NCH):
                    pltpu.async_copy(
                        xb[t].at[pl.ds(c * CH, CH)], acc.at[ib[t].at[c]],
                        sems[t], add=True)
        return carry

    lax.fori_loop(0, NBJ, outer, 0)

    for p in range(4):
        _wait_scatter(p)

    plsc.subcore_barrier()

    @pl.when(sid == 0)
    def _writeout():
        pltpu.sync_copy(acc, out_hbm.at[cid])


def _combine_body(p_ref, o_ref):
    o_ref[...] = p_ref[0] + p_ref[1]


_combine = pl.pallas_call(
    _combine_body,
    out_shape=jax.ShapeDtypeStruct((S, D), jnp.float32),
)


@jax.jit
def kernel(node_feats, batch_idx, W_attn, b_attn, W_mask, b_mask):
    idx = batch_idx.astype(jnp.int32).reshape(N // CH, CH)
    wa = W_attn.reshape(D)
    wm = W_mask.reshape(D)
    bias = jnp.concatenate(
        [b_attn.astype(jnp.float32), b_mask.astype(jnp.float32),
         jnp.zeros((14,), jnp.float32)]
    )
    partials = _sc_pool(node_feats, idx, wa, wm, bias)
    return _combine(partials)
